# baseline (speedup 1.0000x reference)
import functools

import jax
import jax.numpy as jnp
import numpy as np
from jax import lax
from jax.experimental import pallas as pl
from jax.experimental.pallas import tpu as pltpu
from jax.experimental.pallas import tpu_sc as plsc

N = 10000
E = 320000
D = 128
H = 20
C = 2

NP = 10240
WD = 32
GLR = NP * WD // 128
WD2 = 16
GL2R = NP * WD2 // 128

NC = 2
NS = 16
NW = NC * NS
CH = 128
EP = 327680
NSTEP = EP // NW // CH
NBUF = 8
NBUFD = 5
RPT = NP // NS
ORT = GLR // NS
ORT2 = GL2R // NS


def _vmesh():
    return plsc.VectorSubcoreMesh(core_axis_name="c", subcore_axis_name="s")


_SC_PARAMS = pltpu.CompilerParams(use_tc_tiling_on_sc=False)


def _permute_to_wide(narrow, wide, wd, ort, lanes=16):
    npl = wd // lanes
    rpw = 128 // wd
    cpw = 128 // lanes

    @pl.loop(0, ort)
    def _(rr):
        for cc in range(cpw):
            wide[rr, pl.ds(cc * lanes, lanes)] = narrow[
                rr * rpw + cc // npl, pl.ds((cc % npl) * lanes, lanes)
            ]


@jax.jit
def _sc_degree(dst2d):
    @functools.partial(
        pl.kernel,
        out_type=jax.ShapeDtypeStruct((NC, GLR, 128), jnp.float32),
        mesh=_vmesh(),
        compiler_params=_SC_PARAMS,
        scratch_types=[
            pltpu.VMEM((NSTEP, CH), jnp.int32),
            pltpu.VMEM((CH, WD), jnp.float32),
            pltpu.VMEM((RPT, WD), jnp.float32),
            pltpu.VMEM((ORT, 128), jnp.float32),
            pltpu.VMEM_SHARED((NP, WD), jnp.float32),
            pltpu.SemaphoreType.DMA((NBUFD,)),
        ],
    )
    def deg_kernel(dst_hbm, out_hbm, dst_v, ones_v, qbuf, pbuf, acc_sh, sems):
        cid = lax.axis_index("c")
        sid = lax.axis_index("s")
        wid = cid * NS + sid

        pltpu.sync_copy(dst_hbm.at[wid], dst_v)

        ones16 = jnp.ones((16,), jnp.float32)
        zero16 = jnp.zeros((16,), jnp.float32)

        @pl.loop(0, CH)
        def _(r):
            for c in range(WD // 16):
                ones_v[r, pl.ds(c * 16, 16)] = ones16

        @pl.loop(0, RPT)
        def _(r):
            for c in range(WD // 16):
                qbuf[r, pl.ds(c * 16, 16)] = zero16

        pltpu.sync_copy(qbuf, acc_sh.at[pl.ds(sid * RPT, RPT)])
        plsc.subcore_barrier()

        @pl.loop(0, NSTEP, step=NBUFD)
        def _(s0):
            descs = []
            for b in range(NBUFD):
                descs.append(
                    pltpu.async_copy(
                        ones_v, acc_sh.at[dst_v.at[s0 + b]],
                        sems.at[b], add=True,
                    )
                )
            for d in descs:
                d.wait()

        plsc.subcore_barrier()
        pltpu.sync_copy(acc_sh.at[pl.ds(sid * RPT, RPT)], qbuf)
        _permute_to_wide(qbuf, pbuf, WD, ORT)
        pltpu.sync_copy(pbuf, out_hbm.at[cid, pl.ds(sid * ORT, ORT)])

    return deg_kernel(dst2d)


def _make_segsum(wd, glr, ort, dtype=jnp.float32):
  lanes = 32 if dtype == jnp.bfloat16 else 16

  @jax.jit
  def _sc_segsum(gl, src2d, dst2d):
    @functools.partial(
        pl.kernel,
        out_type=jax.ShapeDtypeStruct((NC, glr, 128), dtype),
        mesh=_vmesh(),
        compiler_params=_SC_PARAMS,
        scratch_types=[
            pltpu.VMEM((NSTEP, CH), jnp.int32),
            pltpu.VMEM((NSTEP, CH), jnp.int32),
            pltpu.VMEM((NBUF, CH, wd), dtype),
            pltpu.VMEM((RPT, wd), dtype),
            pltpu.VMEM((ort, 128), dtype),
            pltpu.VMEM_SHARED((NP, wd), dtype),
            pltpu.SemaphoreType.DMA((NBUF,)),
            pltpu.SemaphoreType.DMA((NBUF,)),
        ],
    )
    def seg_kernel(
        g_hbm, src_hbm, dst_hbm, out_hbm,
        src_v, dst_v, rows_v, qbuf, pbuf, acc_sh, gsems, ssems,
    ):
        cid = lax.axis_index("c")
        sid = lax.axis_index("s")
        wid = cid * NS + sid

        pltpu.sync_copy(src_hbm.at[wid], src_v)
        pltpu.sync_copy(dst_hbm.at[wid], dst_v)

        zerov = jnp.zeros((lanes,), dtype)

        @pl.loop(0, RPT)
        def _(r):
            for c in range(wd // lanes):
                qbuf[r, pl.ds(c * lanes, lanes)] = zerov

        pltpu.sync_copy(qbuf, acc_sh.at[pl.ds(sid * RPT, RPT)])
        plsc.subcore_barrier()

        @pl.loop(0, NSTEP, step=NBUF)
        def _(s0):
            gds = []
            for b in range(NBUF):
                gds.append(
                    pltpu.async_copy(
                        g_hbm.at[src_v.at[s0 + b]], rows_v.at[b],
                        gsems.at[b],
                    )
                )
            sds = []
            for b in range(NBUF):
                gds[b].wait()
                sds.append(
                    pltpu.async_copy(
                        rows_v.at[b], acc_sh.at[dst_v.at[s0 + b]],
                        ssems.at[b], add=True,
                    )
                )
            for d in sds:
                d.wait()

        plsc.subcore_barrier()
        pltpu.sync_copy(acc_sh.at[pl.ds(sid * RPT, RPT)], qbuf)
        _permute_to_wide(qbuf, pbuf, wd, ort, lanes)
        pltpu.sync_copy(pbuf, out_hbm.at[cid, pl.ds(sid * ort, ort)])

    return seg_kernel(gl, src2d, dst2d)
  return _sc_segsum


_segsum_l1 = _make_segsum(WD, GLR, ORT)
_segsum_l2 = _make_segsum(WD2, GL2R, ORT2)


def _tc_h1(x4, w1bd):
    def body(x_ref, w_ref, h_ref):
        h_ref[...] = jnp.dot(
            x_ref[...], w_ref[...], precision=lax.Precision.HIGHEST
        )

    return pl.pallas_call(
        body, out_shape=jax.ShapeDtypeStruct((GLR, 128), jnp.float32)
    )(x4, w1bd)


def _tc_g1(h1l, degp):
    def body(h_ref, d_ref, g_ref):
        dis = lax.rsqrt(d_ref[0] + d_ref[1] + 1.0)
        g_ref[...] = h_ref[...] * dis

    return pl.pallas_call(
        body, out_shape=jax.ShapeDtypeStruct((GLR, 128), jnp.float32)
    )(h1l, degp)


def _tc_stage2(s1p, g1l, degp, b1bc, w2bd8):
    def body(s_ref, g_ref, d_ref, b_ref, w_ref, o_ref):
        dis = lax.rsqrt(d_ref[0] + d_ref[1] + 1.0)
        pre = (s_ref[0] + s_ref[1] + g_ref[...]) * dis + b_ref[...]
        act = jnp.where(pre >= 0, pre, 0.01 * pre)
        act2 = jnp.reshape(act * dis, (GL2R, 256))
        o_ref[...] = jnp.dot(
            act2, w_ref[...], precision=lax.Precision.HIGHEST
        )

    return pl.pallas_call(
        body, out_shape=jax.ShapeDtypeStruct((GL2R, 128), jnp.float32)
    )(s1p, g1l, degp, b1bc, w2bd8)


def _tc_stage3(s2p, g2l, degp, b2bc, sel_a, sel_b, dsel):
    def body(s_ref, g_ref, d_ref, b_ref, sa_ref, sb_ref, ds_ref, o_ref):
        dp2 = jnp.reshape(d_ref[0] + d_ref[1] + 1.0, (GL2R, 256))
        dp16 = jnp.dot(dp2, ds_ref[...], precision=lax.Precision.HIGHEST)
        dis = lax.rsqrt(dp16)
        z = (s_ref[0] + s_ref[1] + g_ref[...]) * dis + b_ref[...]
        za = jnp.dot(z, sa_ref[...], precision=lax.Precision.HIGHEST)
        zb = jnp.dot(z, sb_ref[...], precision=lax.Precision.HIGHEST)
        m = jnp.maximum(za, zb)
        lse = m + jnp.log(jnp.exp(za - m) + jnp.exp(zb - m))
        o_ref[...] = za - lse

    return pl.pallas_call(
        body, out_shape=jax.ShapeDtypeStruct((GL2R, 16), jnp.float32)
    )(s2p, g2l, degp, b2bc, sel_a, sel_b, dsel)


_DSEL = np.zeros((256, 128), np.float32)
for _j in range(8):
    for _c in range(16):
        _DSEL[32 * _j + _c, 16 * _j + _c] = 1.0

_SEL_A = np.zeros((128, 16), np.float32)
_SEL_B = np.zeros((128, 16), np.float32)
for _j in range(8):
    _SEL_A[16 * _j + 0, 2 * _j + 0] = 1.0
    _SEL_A[16 * _j + 1, 2 * _j + 1] = 1.0
    _SEL_B[16 * _j + 1, 2 * _j + 0] = 1.0
    _SEL_B[16 * _j + 0, 2 * _j + 1] = 1.0


@jax.jit
def kernel(x, edge_index, W1, b1, W2, b2):
    pad_idx = N + jnp.arange(EP - E, dtype=jnp.int32) % (NP - N)
    dst2d = jnp.concatenate([edge_index[1], pad_idx]).reshape(NW, NSTEP, CH)
    src2d = jnp.concatenate([edge_index[0], pad_idx]).reshape(NW, NSTEP, CH)

    x4 = jnp.pad(x, ((0, NP - N), (0, 0))).reshape(GLR, 4 * D)
    w1p = jnp.pad(W1, ((0, 0), (0, WD - H)))
    w1bd = jnp.kron(jnp.eye(4, dtype=jnp.float32), w1p)
    w2p = jnp.pad(W2, ((0, WD - H), (0, WD2 - C)))
    w2bd8 = jnp.kron(jnp.eye(8, dtype=jnp.float32), w2p)
    b1bc = jnp.tile(jnp.pad(b1, (0, WD - H)), 4).reshape(1, 128)
    b2bc = jnp.tile(jnp.pad(b2, (0, WD2 - C)), 8).reshape(1, 128)
    sel_a = jnp.asarray(_SEL_A)
    sel_b = jnp.asarray(_SEL_B)
    dsel = jnp.asarray(_DSEL)

    h1l = _tc_h1(x4, w1bd)
    degp = _sc_degree(dst2d)
    g1l = _tc_g1(h1l, degp)
    s1p = _segsum_l1(g1l.reshape(NP, WD), src2d, dst2d)
    g2l = _tc_stage2(s1p, g1l, degp, b1bc, w2bd8)
    s2p = _segsum_l2(g2l.reshape(NP, WD2), src2d, dst2d)
    out16 = _tc_stage3(s2p, g2l, degp, b2bc, sel_a, sel_b, dsel)
    return out16.reshape(NP, C)[:N]


# speedup vs baseline: 1.0443x; 1.0443x over previous
import functools

import jax
import jax.numpy as jnp
import numpy as np
from jax import lax
from jax.experimental import pallas as pl
from jax.experimental.pallas import tpu as pltpu
from jax.experimental.pallas import tpu_sc as plsc

N = 10000
E = 320000
D = 128
H = 20
C = 2

NP = 10240
WD = 32
GLR = NP * WD // 128
WD2 = 16
GL2R = NP * WD2 // 128

NC = 2
NS = 16
NW = NC * NS
CH = 128
EP = 327680
NSTEP = EP // NW // CH
NBUF = 8
NBUFD = 5
RPT = NP // NS
ORT = GLR // NS
ORT2 = GL2R // NS


def _vmesh():
    return plsc.VectorSubcoreMesh(core_axis_name="c", subcore_axis_name="s")


_SC_PARAMS = pltpu.CompilerParams(use_tc_tiling_on_sc=False)


def _permute_to_wide(narrow, wide, wd, ort, lanes=16):
    npl = wd // lanes
    rpw = 128 // wd
    cpw = 128 // lanes

    @pl.loop(0, ort)
    def _(rr):
        for cc in range(cpw):
            wide[rr, pl.ds(cc * lanes, lanes)] = narrow[
                rr * rpw + cc // npl, pl.ds((cc % npl) * lanes, lanes)
            ]


@jax.jit
def _sc_degree(dst2d):
    @functools.partial(
        pl.kernel,
        out_type=jax.ShapeDtypeStruct((NC, GLR, 128), jnp.float32),
        mesh=_vmesh(),
        compiler_params=_SC_PARAMS,
        scratch_types=[
            pltpu.VMEM((NSTEP, CH), jnp.int32),
            pltpu.VMEM((CH, WD), jnp.float32),
            pltpu.VMEM((RPT, WD), jnp.float32),
            pltpu.VMEM((ORT, 128), jnp.float32),
            pltpu.VMEM_SHARED((NP, WD), jnp.float32),
            pltpu.SemaphoreType.DMA((NBUFD,)),
        ],
    )
    def deg_kernel(dst_hbm, out_hbm, dst_v, ones_v, qbuf, pbuf, acc_sh, sems):
        cid = lax.axis_index("c")
        sid = lax.axis_index("s")
        wid = cid * NS + sid

        pltpu.sync_copy(dst_hbm.at[wid], dst_v)

        ones16 = jnp.ones((16,), jnp.float32)
        zero16 = jnp.zeros((16,), jnp.float32)

        @pl.loop(0, CH)
        def _(r):
            for c in range(WD // 16):
                ones_v[r, pl.ds(c * 16, 16)] = ones16

        @pl.loop(0, RPT)
        def _(r):
            for c in range(WD // 16):
                qbuf[r, pl.ds(c * 16, 16)] = zero16

        pltpu.sync_copy(qbuf, acc_sh.at[pl.ds(sid * RPT, RPT)])
        plsc.subcore_barrier()

        @pl.loop(0, NSTEP, step=NBUFD)
        def _(s0):
            descs = []
            for b in range(NBUFD):
                descs.append(
                    pltpu.async_copy(
                        ones_v, acc_sh.at[dst_v.at[s0 + b]],
                        sems.at[b], add=True,
                    )
                )
            for d in descs:
                d.wait()

        plsc.subcore_barrier()
        pltpu.sync_copy(acc_sh.at[pl.ds(sid * RPT, RPT)], qbuf)
        _permute_to_wide(qbuf, pbuf, WD, ORT)
        pltpu.sync_copy(pbuf, out_hbm.at[cid, pl.ds(sid * ORT, ORT)])

    return deg_kernel(dst2d)


def _make_segsum(wd, glr, ort, dtype=jnp.float32):
  lanes = 32 if dtype == jnp.bfloat16 else 16

  @jax.jit
  def _sc_segsum(gl, src2d, dst2d):
    @functools.partial(
        pl.kernel,
        out_type=jax.ShapeDtypeStruct((NC, glr, 128), dtype),
        mesh=_vmesh(),
        compiler_params=_SC_PARAMS,
        scratch_types=[
            pltpu.VMEM((NSTEP, CH), jnp.int32),
            pltpu.VMEM((NSTEP, CH), jnp.int32),
            pltpu.VMEM((NBUF, CH, wd), dtype),
            pltpu.VMEM((RPT, wd), dtype),
            pltpu.VMEM((ort, 128), dtype),
            pltpu.VMEM_SHARED((NP, wd), dtype),
            pltpu.SemaphoreType.DMA((NBUF,)),
            pltpu.SemaphoreType.DMA((NBUF,)),
        ],
    )
    def seg_kernel(
        g_hbm, src_hbm, dst_hbm, out_hbm,
        src_v, dst_v, rows_v, qbuf, pbuf, acc_sh, gsems, ssems,
    ):
        cid = lax.axis_index("c")
        sid = lax.axis_index("s")
        wid = cid * NS + sid

        pltpu.sync_copy(src_hbm.at[wid], src_v)
        pltpu.sync_copy(dst_hbm.at[wid], dst_v)

        zerov = jnp.zeros((lanes,), dtype)

        @pl.loop(0, RPT)
        def _(r):
            for c in range(wd // lanes):
                qbuf[r, pl.ds(c * lanes, lanes)] = zerov

        pltpu.sync_copy(qbuf, acc_sh.at[pl.ds(sid * RPT, RPT)])
        plsc.subcore_barrier()

        @pl.loop(0, NSTEP, step=NBUF)
        def _(s0):
            gds = []
            for b in range(NBUF):
                gds.append(
                    pltpu.async_copy(
                        g_hbm.at[src_v.at[s0 + b]], rows_v.at[b],
                        gsems.at[b],
                    )
                )
            sds = []
            for b in range(NBUF):
                gds[b].wait()
                sds.append(
                    pltpu.async_copy(
                        rows_v.at[b], acc_sh.at[dst_v.at[s0 + b]],
                        ssems.at[b], add=True,
                    )
                )
            for d in sds:
                d.wait()

        plsc.subcore_barrier()
        pltpu.sync_copy(acc_sh.at[pl.ds(sid * RPT, RPT)], qbuf)
        _permute_to_wide(qbuf, pbuf, wd, ort, lanes)
        pltpu.sync_copy(pbuf, out_hbm.at[cid, pl.ds(sid * ort, ort)])

    return seg_kernel(gl, src2d, dst2d)
  return _sc_segsum


_segsum_l1 = _make_segsum(WD, GLR, ORT)
_segsum_l2 = _make_segsum(WD2, GL2R, ORT2)


def _tc_h1(x4, w1bd):
    def body(x_ref, w_ref, h_ref):
        h_ref[...] = jnp.dot(
            x_ref[...], w_ref[...], precision=lax.Precision.HIGHEST
        )

    return pl.pallas_call(
        body, out_shape=jax.ShapeDtypeStruct((GLR, 128), jnp.float32)
    )(x4, w1bd)


def _tc_g1(h1l, degp):
    def body(h_ref, d_ref, g_ref):
        dis = lax.rsqrt(d_ref[0] + d_ref[1] + 1.0)
        g_ref[...] = h_ref[...] * dis

    return pl.pallas_call(
        body, out_shape=jax.ShapeDtypeStruct((GLR, 128), jnp.float32)
    )(h1l, degp)


def _tc_stage2(s1p, g1l, degp, b1bc, w2bd8):
    def body(s_ref, g_ref, d_ref, b_ref, w_ref, o_ref):
        dis = lax.rsqrt(d_ref[0] + d_ref[1] + 1.0)
        pre = (s_ref[0] + s_ref[1] + g_ref[...]) * dis + b_ref[...]
        act = jnp.where(pre >= 0, pre, 0.01 * pre)
        act2 = jnp.reshape(act * dis, (GL2R, 256))
        o_ref[...] = jnp.dot(
            act2, w_ref[...], precision=lax.Precision.HIGHEST
        )

    return pl.pallas_call(
        body, out_shape=jax.ShapeDtypeStruct((GL2R, 128), jnp.float32)
    )(s1p, g1l, degp, b1bc, w2bd8)


def _tc_stage3(s2p, g2l, degp, b2bc, sel_a, sel_b, dsel):
    def body(s_ref, g_ref, d_ref, b_ref, sa_ref, sb_ref, ds_ref, o_ref):
        dp2 = jnp.reshape(d_ref[0] + d_ref[1] + 1.0, (GL2R, 256))
        dp16 = jnp.dot(dp2, ds_ref[...], precision=lax.Precision.HIGHEST)
        dis = lax.rsqrt(dp16)
        z = (s_ref[0] + s_ref[1] + g_ref[...]) * dis + b_ref[...]
        za = jnp.dot(z, sa_ref[...], precision=lax.Precision.HIGHEST)
        zb = jnp.dot(z, sb_ref[...], precision=lax.Precision.HIGHEST)
        m = jnp.maximum(za, zb)
        lse = m + jnp.log(jnp.exp(za - m) + jnp.exp(zb - m))
        o_ref[...] = za - lse

    return pl.pallas_call(
        body, out_shape=jax.ShapeDtypeStruct((GL2R, 16), jnp.float32)
    )(s2p, g2l, degp, b2bc, sel_a, sel_b, dsel)


_DSEL = np.zeros((256, 128), np.float32)
for _j in range(8):
    for _c in range(16):
        _DSEL[32 * _j + _c, 16 * _j + _c] = 1.0

_SEL_A = np.zeros((128, 16), np.float32)
_SEL_B = np.zeros((128, 16), np.float32)
for _j in range(8):
    _SEL_A[16 * _j + 0, 2 * _j + 0] = 1.0
    _SEL_A[16 * _j + 1, 2 * _j + 1] = 1.0
    _SEL_B[16 * _j + 1, 2 * _j + 0] = 1.0
    _SEL_B[16 * _j + 0, 2 * _j + 1] = 1.0


@jax.jit
def kernel(x, edge_index, W1, b1, W2, b2):
    pad_idx = N + jnp.arange(EP - E, dtype=jnp.int32) % (NP - N)
    ep = jnp.concatenate([edge_index, jnp.stack([pad_idx, pad_idx])], axis=1)
    src2d = ep[0].reshape(NW, NSTEP, CH)
    dst2d = ep[1].reshape(NW, NSTEP, CH)

    x4 = jnp.pad(x, ((0, NP - N), (0, 0))).reshape(GLR, 4 * D)
    w1p = jnp.pad(W1, ((0, 0), (0, WD - H)))
    w1bd = jnp.kron(jnp.eye(4, dtype=jnp.float32), w1p)
    w2p = jnp.pad(W2, ((0, WD - H), (0, WD2 - C)))
    w2bd8 = jnp.kron(jnp.eye(8, dtype=jnp.float32), w2p)
    b1bc = jnp.tile(jnp.pad(b1, (0, WD - H)), 4).reshape(1, 128)
    b2bc = jnp.tile(jnp.pad(b2, (0, WD2 - C)), 8).reshape(1, 128)
    sel_a = jnp.asarray(_SEL_A)
    sel_b = jnp.asarray(_SEL_B)
    dsel = jnp.asarray(_DSEL)

    h1l = _tc_h1(x4, w1bd)
    degp = _sc_degree(dst2d)
    g1l = _tc_g1(h1l, degp)
    s1p = _segsum_l1(g1l.reshape(NP, WD), src2d, dst2d)
    g2l = _tc_stage2(s1p, g1l, degp, b1bc, w2bd8)
    s2p = _segsum_l2(g2l.reshape(NP, WD2), src2d, dst2d)
    out16 = _tc_stage3(s2p, g2l, degp, b2bc, sel_a, sel_b, dsel)
    return out16.reshape(NP, C)[:N]


# speedup vs baseline: 1.0711x; 1.0257x over previous
import functools

import jax
import jax.numpy as jnp
import numpy as np
from jax import lax
from jax.experimental import pallas as pl
from jax.experimental.pallas import tpu as pltpu
from jax.experimental.pallas import tpu_sc as plsc

N = 10000
E = 320000
D = 128
H = 20
C = 2

NP = 10240
WD = 32
GLR = NP * WD // 128
WD2 = 16
GL2R = NP * WD2 // 128

NC = 2
NS = 16
NW = NC * NS
CH = 128
EP = 327680
NSTEP = EP // NW // CH
NBUF = 10
NBUFD = 5
RPT = NP // NS
ORT = GLR // NS
ORT2 = GL2R // NS


def _vmesh():
    return plsc.VectorSubcoreMesh(core_axis_name="c", subcore_axis_name="s")


_SC_PARAMS = pltpu.CompilerParams(use_tc_tiling_on_sc=False)


def _permute_to_wide(narrow, wide, wd, ort, lanes=16):
    npl = wd // lanes
    rpw = 128 // wd
    cpw = 128 // lanes

    @pl.loop(0, ort)
    def _(rr):
        for cc in range(cpw):
            wide[rr, pl.ds(cc * lanes, lanes)] = narrow[
                rr * rpw + cc // npl, pl.ds((cc % npl) * lanes, lanes)
            ]


@jax.jit
def _sc_degree(dst2d):
    @functools.partial(
        pl.kernel,
        out_type=jax.ShapeDtypeStruct((NC, GLR, 128), jnp.float32),
        mesh=_vmesh(),
        compiler_params=_SC_PARAMS,
        scratch_types=[
            pltpu.VMEM((NSTEP, CH), jnp.int32),
            pltpu.VMEM((CH, WD), jnp.float32),
            pltpu.VMEM((RPT, WD), jnp.float32),
            pltpu.VMEM((ORT, 128), jnp.float32),
            pltpu.VMEM_SHARED((NP, WD), jnp.float32),
            pltpu.SemaphoreType.DMA((NBUFD,)),
        ],
    )
    def deg_kernel(dst_hbm, out_hbm, dst_v, ones_v, qbuf, pbuf, acc_sh, sems):
        cid = lax.axis_index("c")
        sid = lax.axis_index("s")
        wid = cid * NS + sid

        pltpu.sync_copy(dst_hbm.at[wid], dst_v)

        ones16 = jnp.ones((16,), jnp.float32)
        zero16 = jnp.zeros((16,), jnp.float32)

        @pl.loop(0, CH)
        def _(r):
            for c in range(WD // 16):
                ones_v[r, pl.ds(c * 16, 16)] = ones16

        @pl.loop(0, RPT)
        def _(r):
            for c in range(WD // 16):
                qbuf[r, pl.ds(c * 16, 16)] = zero16

        pltpu.sync_copy(qbuf, acc_sh.at[pl.ds(sid * RPT, RPT)])
        plsc.subcore_barrier()

        @pl.loop(0, NSTEP, step=NBUFD)
        def _(s0):
            descs = []
            for b in range(NBUFD):
                descs.append(
                    pltpu.async_copy(
                        ones_v, acc_sh.at[dst_v.at[s0 + b]],
                        sems.at[b], add=True,
                    )
                )
            for d in descs:
                d.wait()

        plsc.subcore_barrier()
        pltpu.sync_copy(acc_sh.at[pl.ds(sid * RPT, RPT)], qbuf)
        _permute_to_wide(qbuf, pbuf, WD, ORT)
        pltpu.sync_copy(pbuf, out_hbm.at[cid, pl.ds(sid * ORT, ORT)])

    return deg_kernel(dst2d)


def _make_segsum(wd, glr, ort, dtype=jnp.float32):
  lanes = 32 if dtype == jnp.bfloat16 else 16

  @jax.jit
  def _sc_segsum(gl, src2d, dst2d):
    @functools.partial(
        pl.kernel,
        out_type=jax.ShapeDtypeStruct((NC, glr, 128), dtype),
        mesh=_vmesh(),
        compiler_params=_SC_PARAMS,
        scratch_types=[
            pltpu.VMEM((NSTEP, CH), jnp.int32),
            pltpu.VMEM((NSTEP, CH), jnp.int32),
            pltpu.VMEM((NBUF, CH, wd), dtype),
            pltpu.VMEM((RPT, wd), dtype),
            pltpu.VMEM((ort, 128), dtype),
            pltpu.VMEM_SHARED((NP, wd), dtype),
            pltpu.SemaphoreType.DMA((NBUF,)),
            pltpu.SemaphoreType.DMA((NBUF,)),
        ],
    )
    def seg_kernel(
        g_hbm, src_hbm, dst_hbm, out_hbm,
        src_v, dst_v, rows_v, qbuf, pbuf, acc_sh, gsems, ssems,
    ):
        cid = lax.axis_index("c")
        sid = lax.axis_index("s")
        wid = cid * NS + sid

        pltpu.sync_copy(src_hbm.at[wid], src_v)
        pltpu.sync_copy(dst_hbm.at[wid], dst_v)

        zerov = jnp.zeros((lanes,), dtype)

        @pl.loop(0, RPT)
        def _(r):
            for c in range(wd // lanes):
                qbuf[r, pl.ds(c * lanes, lanes)] = zerov

        pltpu.sync_copy(qbuf, acc_sh.at[pl.ds(sid * RPT, RPT)])
        plsc.subcore_barrier()

        @pl.loop(0, NSTEP, step=NBUF)
        def _(s0):
            gds = []
            for b in range(NBUF):
                gds.append(
                    pltpu.async_copy(
                        g_hbm.at[src_v.at[s0 + b]], rows_v.at[b],
                        gsems.at[b],
                    )
                )
            sds = []
            for b in range(NBUF):
                gds[b].wait()
                sds.append(
                    pltpu.async_copy(
                        rows_v.at[b], acc_sh.at[dst_v.at[s0 + b]],
                        ssems.at[b], add=True,
                    )
                )
            for d in sds:
                d.wait()

        plsc.subcore_barrier()
        pltpu.sync_copy(acc_sh.at[pl.ds(sid * RPT, RPT)], qbuf)
        _permute_to_wide(qbuf, pbuf, wd, ort, lanes)
        pltpu.sync_copy(pbuf, out_hbm.at[cid, pl.ds(sid * ort, ort)])

    return seg_kernel(gl, src2d, dst2d)
  return _sc_segsum


_segsum_l1 = _make_segsum(WD, GLR, ORT)
_segsum_l2 = _make_segsum(WD2, GL2R, ORT2)


def _tc_h1(x4, w1bd):
    def body(x_ref, w_ref, h_ref):
        h_ref[...] = jnp.dot(
            x_ref[...], w_ref[...], precision=lax.Precision.HIGHEST
        )

    return pl.pallas_call(
        body, out_shape=jax.ShapeDtypeStruct((GLR, 128), jnp.float32)
    )(x4, w1bd)


def _tc_g1(h1l, degp):
    def body(h_ref, d_ref, g_ref):
        dis = lax.rsqrt(d_ref[0] + d_ref[1] + 1.0)
        g_ref[...] = h_ref[...] * dis

    return pl.pallas_call(
        body, out_shape=jax.ShapeDtypeStruct((GLR, 128), jnp.float32)
    )(h1l, degp)


def _tc_stage2(s1p, g1l, degp, b1bc, w2bd8):
    def body(s_ref, g_ref, d_ref, b_ref, w_ref, o_ref):
        dis = lax.rsqrt(d_ref[0] + d_ref[1] + 1.0)
        pre = (s_ref[0] + s_ref[1] + g_ref[...]) * dis + b_ref[...]
        act = jnp.where(pre >= 0, pre, 0.01 * pre)
        act2 = jnp.reshape(act * dis, (GL2R, 256))
        o_ref[...] = jnp.dot(
            act2, w_ref[...], precision=lax.Precision.HIGHEST
        )

    return pl.pallas_call(
        body, out_shape=jax.ShapeDtypeStruct((GL2R, 128), jnp.float32)
    )(s1p, g1l, degp, b1bc, w2bd8)


def _tc_stage3(s2p, g2l, degp, b2bc, sel_a, sel_b, dsel):
    def body(s_ref, g_ref, d_ref, b_ref, sa_ref, sb_ref, ds_ref, o_ref):
        dp2 = jnp.reshape(d_ref[0] + d_ref[1] + 1.0, (GL2R, 256))
        dp16 = jnp.dot(dp2, ds_ref[...], precision=lax.Precision.HIGHEST)
        dis = lax.rsqrt(dp16)
        z = (s_ref[0] + s_ref[1] + g_ref[...]) * dis + b_ref[...]
        za = jnp.dot(z, sa_ref[...], precision=lax.Precision.HIGHEST)
        zb = jnp.dot(z, sb_ref[...], precision=lax.Precision.HIGHEST)
        m = jnp.maximum(za, zb)
        lse = m + jnp.log(jnp.exp(za - m) + jnp.exp(zb - m))
        o_ref[...] = za - lse

    return pl.pallas_call(
        body, out_shape=jax.ShapeDtypeStruct((GL2R, 16), jnp.float32)
    )(s2p, g2l, degp, b2bc, sel_a, sel_b, dsel)


_DSEL = np.zeros((256, 128), np.float32)
for _j in range(8):
    for _c in range(16):
        _DSEL[32 * _j + _c, 16 * _j + _c] = 1.0

_SEL_A = np.zeros((128, 16), np.float32)
_SEL_B = np.zeros((128, 16), np.float32)
for _j in range(8):
    _SEL_A[16 * _j + 0, 2 * _j + 0] = 1.0
    _SEL_A[16 * _j + 1, 2 * _j + 1] = 1.0
    _SEL_B[16 * _j + 1, 2 * _j + 0] = 1.0
    _SEL_B[16 * _j + 0, 2 * _j + 1] = 1.0


@jax.jit
def kernel(x, edge_index, W1, b1, W2, b2):
    pad_idx = N + jnp.arange(EP - E, dtype=jnp.int32) % (NP - N)
    ep = jnp.concatenate([edge_index, jnp.stack([pad_idx, pad_idx])], axis=1)
    src2d = ep[0].reshape(NW, NSTEP, CH)
    dst2d = ep[1].reshape(NW, NSTEP, CH)

    x4 = jnp.pad(x, ((0, NP - N), (0, 0))).reshape(GLR, 4 * D)
    w1p = jnp.pad(W1, ((0, 0), (0, WD - H)))
    w1bd = jnp.kron(jnp.eye(4, dtype=jnp.float32), w1p)
    w2p = jnp.pad(W2, ((0, WD - H), (0, WD2 - C)))
    w2bd8 = jnp.kron(jnp.eye(8, dtype=jnp.float32), w2p)
    b1bc = jnp.tile(jnp.pad(b1, (0, WD - H)), 4).reshape(1, 128)
    b2bc = jnp.tile(jnp.pad(b2, (0, WD2 - C)), 8).reshape(1, 128)
    sel_a = jnp.asarray(_SEL_A)
    sel_b = jnp.asarray(_SEL_B)
    dsel = jnp.asarray(_DSEL)

    h1l = _tc_h1(x4, w1bd)
    degp = _sc_degree(dst2d)
    g1l = _tc_g1(h1l, degp)
    s1p = _segsum_l1(g1l.reshape(NP, WD), src2d, dst2d)
    g2l = _tc_stage2(s1p, g1l, degp, b1bc, w2bd8)
    s2p = _segsum_l2(g2l.reshape(NP, WD2), src2d, dst2d)
    out16 = _tc_stage3(s2p, g2l, degp, b2bc, sel_a, sel_b, dsel)
    return out16.reshape(NP, C)[:N]
